# baseline (speedup 1.0000x reference)
import jax
import jax.numpy as jnp
from jax.experimental import pallas as pl
from jax.experimental.pallas import tpu as pltpu

_B = 64
_C = 768
_H = 32
_W = 32
_CB = 96
_NCB = _C // _CB


def _permute_body(idx_ref, x_ref, o_ref):
    jc = pl.program_id(1)
    for i in range(_CB):
        o_ref[0, i] = x_ref[0, idx_ref[jc * _CB + i]]


def _tc_shuffle(x, indices):
    grid_spec = pltpu.PrefetchScalarGridSpec(
        num_scalar_prefetch=1,
        grid=(_B, _NCB),
        in_specs=[
            pl.BlockSpec((1, _C, _H, _W), lambda b, jc, idx_ref: (b, 0, 0, 0)),
        ],
        out_specs=pl.BlockSpec(
            (1, _CB, _H, _W), lambda b, jc, idx_ref: (b, jc, 0, 0)
        ),
    )
    return pl.pallas_call(
        _permute_body,
        grid_spec=grid_spec,
        out_shape=jax.ShapeDtypeStruct((_B, _C, _H, _W), jnp.float32),
    )(indices, x)


def kernel(x, objective, z_list, indices):
    y = _tc_shuffle(x, indices)
    return (y, objective, z_list)


# speedup vs baseline: 1.6842x; 1.6842x over previous
import functools

import jax
import jax.numpy as jnp
from jax import lax
from jax.experimental import pallas as pl
from jax.experimental.pallas import tpu as pltpu
from jax.experimental.pallas import tpu_sc as plsc

_B = 64
_C = 768
_HW = 1024
_R = _B * _C
_NC = 2
_NS = 16
_NW = _NC * _NS
_RPW = _R // _NW
_CH = 48
_NCH = _RPW // _CH
_NBUF = 2


def _sc_gather(xf, idx3):
    mesh = plsc.VectorSubcoreMesh(core_axis_name="c", subcore_axis_name="s")

    @functools.partial(
        pl.kernel,
        mesh=mesh,
        out_type=jax.ShapeDtypeStruct((_R, _HW), jnp.float32),
        compiler_params=pltpu.CompilerParams(use_tc_tiling_on_sc=True),
        scratch_types=[
            pltpu.VMEM((_NCH, _CH), jnp.int32),
            pltpu.VMEM((_CH, _HW), jnp.float32),
            pltpu.VMEM((_CH, _HW), jnp.float32),
            pltpu.SemaphoreType.DMA,
            pltpu.SemaphoreType.DMA,
            pltpu.SemaphoreType.DMA,
            pltpu.SemaphoreType.DMA,
        ],
    )
    def k(x_hbm, idx_hbm, out_hbm, idx_v, rows0, rows1, g0, g1, s0, s1):
        wid = lax.axis_index("s") * _NC + lax.axis_index("c")
        pltpu.sync_copy(idx_hbm.at[wid], idx_v)
        base = wid * _RPW
        rows = (rows0, rows1)
        gsem = (g0, g1)
        ssem = (s0, s1)

        def start_gather(j):
            p = j % _NBUF
            return pltpu.async_copy(x_hbm.at[idx_v.at[j]], rows[p], gsem[p])

        def start_scatter(j):
            p = j % _NBUF
            return pltpu.async_copy(
                rows[p], out_hbm.at[pl.ds(base + j * _CH, _CH)], ssem[p]
            )

        g = {0: start_gather(0), 1: start_gather(1)}
        s = {}
        for j in range(_NCH):
            g[j].wait()
            s[j] = start_scatter(j)
            if j + _NBUF < _NCH:
                s[j].wait()
                g[j + _NBUF] = start_gather(j + _NBUF)
        for j in range(_NCH - _NBUF, _NCH):
            s[j].wait()

    return k(xf, idx3)


def kernel(x, objective, z_list, indices):
    xf = x.reshape(_R, _HW)
    idx3 = (
        jnp.arange(_B, dtype=jnp.int32)[:, None] * _C + indices[None, :]
    ).reshape(_NW, _NCH, _CH)
    yf = _sc_gather(xf, idx3)
    return (yf.reshape(_B, _C, 32, 32), objective, z_list)
